# Initial kernel scaffold; baseline (speedup 1.0000x reference)
import jax
import jax.numpy as jnp
from jax.experimental import pallas as pl


def kernel(inputs, knn, W_res, b_res, W1, b1, Wl1, bl1, Wl2, bl2, W2, b2):
    raise NotImplementedError("write your pallas kernel here")


# speedup vs baseline: 4.2056x; 4.2056x over previous
import functools

import jax
import jax.numpy as jnp
from jax import lax
from jax.experimental import pallas as pl
from jax.experimental.pallas import tpu as pltpu
from jax.experimental.pallas import tpu_sc as plsc

NC = 2
NS = 16
NW = NC * NS


def _sc_gather(table, idx):
    M = idx.shape[0]
    T, W = table.shape
    b_per_w = M // NW
    n_chunk = 5
    chunk = b_per_w // n_chunk
    mesh = plsc.VectorSubcoreMesh(
        core_axis_name="c", subcore_axis_name="s", num_cores=NC, num_subcores=NS
    )

    @functools.partial(
        pl.kernel,
        mesh=mesh,
        out_type=jax.ShapeDtypeStruct((M, W), jnp.float32),
        scratch_types=[
            pltpu.VMEM((chunk,), jnp.int32),
            pltpu.VMEM((chunk, W), jnp.float32),
            pltpu.SemaphoreType.DMA,
        ],
        compiler_params=pltpu.CompilerParams(use_tc_tiling_on_sc=False),
    )
    def k(table_hbm, idx_hbm, out_hbm, idx_v, rows_v, sem):
        wid = lax.axis_index("s") * NC + lax.axis_index("c")
        base = wid * b_per_w
        for ci in range(n_chunk):
            off = base + ci * chunk
            pltpu.sync_copy(idx_hbm.at[pl.ds(off, chunk)], idx_v)
            pltpu.async_copy(table_hbm.at[idx_v], rows_v, sem).wait()
            pltpu.sync_copy(rows_v, out_hbm.at[pl.ds(off, chunk)])

    return k(table, idx)


def _tc_call(body, grid, in_specs, out_shapes, out_specs):
    return pl.pallas_call(
        body,
        grid=grid,
        in_specs=in_specs,
        out_shape=out_shapes,
        out_specs=out_specs,
    )


def _full(a):
    return pl.BlockSpec(a.shape, lambda i: (0,) * a.ndim)


_DOT = functools.partial(jnp.dot, precision=jax.lax.Precision.HIGHEST)


def kernel(inputs, knn, W_res, b_res, W1, b1, Wl1, bl1, Wl2, bl2, W2, b2):
    N, K = knn.shape
    d = W_res.shape[0]
    B = 1000
    G = N // B

    idx = knn.astype(jnp.int32).T.reshape(-1)

    xpad = jnp.pad(inputs, ((0, 0), (0, 13)))

    Wl1p = jnp.pad(Wl1[0:3], ((0, 13), (0, 0)))
    Wl2p = jnp.pad(Wl2[0:3], ((0, 13), (0, 0)))
    Wl1r3 = Wl1[3:4]
    Wl2r3 = Wl2[3:4]
    bl1r = bl1[None, :]
    bl2r = bl2[None, :]
    b1r = b1[None, :]
    b2r = b2[None, :]
    b_resr = b_res[None, :]

    g0 = _sc_gather(xpad, idx).reshape(K, N, 16)

    def gf_body(x_ref, g_ref, wl1_ref, wl2_ref, wr1_ref, wr2_ref,
                bl1_ref, bl2_ref, a1_ref, a2_ref):
        x = x_ref[...]
        dsum = jnp.zeros((B, 16), jnp.float32)
        nsum = jnp.zeros((B, 1), jnp.float32)
        for k in range(K):
            dk = x - g_ref[k]
            dsum = dsum + dk
            nsum = nsum + jnp.sqrt(jnp.sum(dk * dk, axis=1, keepdims=True))
        inv_k = 1.0 / K
        a1_ref[...] = (_DOT(dsum, wl1_ref[...]) + nsum * wr1_ref[...]) * inv_k \
            + bl1_ref[...]
        a2_ref[...] = (_DOT(dsum, wl2_ref[...]) + nsum * wr2_ref[...]) * inv_k \
            + bl2_ref[...]

    a1, a2 = _tc_call(
        gf_body,
        grid=(G,),
        in_specs=[
            pl.BlockSpec((B, 16), lambda i: (i, 0)),
            pl.BlockSpec((K, B, 16), lambda i: (0, i, 0)),
            _full(Wl1p), _full(Wl2p), _full(Wl1r3), _full(Wl2r3),
            _full(bl1r), _full(bl2r),
        ],
        out_shapes=(
            jax.ShapeDtypeStruct((N, 32), jnp.float32),
            jax.ShapeDtypeStruct((N, 64), jnp.float32),
        ),
        out_specs=(
            pl.BlockSpec((B, 32), lambda i: (i, 0)),
            pl.BlockSpec((B, 64), lambda i: (i, 0)),
        ),
    )(xpad, g0, Wl1p, Wl2p, Wl1r3, Wl2r3, bl1r, bl2r)

    g1 = _sc_gather(a1, idx).reshape(K, N, 32)

    def c_body(g_ref, a2_ref, w2_ref, w1_ref, b2_ref, b1_ref, bres_ref,
               c_ref, f1_ref, x1_ref):
        psum = jnp.zeros((B, 32), jnp.float32)
        for k in range(K):
            psum = psum + g_ref[k]
        p = psum * (1.0 / K)
        w2 = w2_ref[...]
        c = _DOT(a2_ref[...], w2[0:64, :]) + _DOT(p, w2[64:96, :]) + b2_ref[...]
        row = _DOT(b1_ref[...], w2[96:128, :]) + bres_ref[...]
        f1 = c + row
        c_ref[...] = c
        f1_ref[...] = f1
        x1_ref[...] = _DOT(f1, w1_ref[...]) + b1_ref[...]

    c, f1, x1 = _tc_call(
        c_body,
        grid=(G,),
        in_specs=[
            pl.BlockSpec((K, B, 32), lambda i: (0, i, 0)),
            pl.BlockSpec((B, 64), lambda i: (i, 0)),
            _full(W2), _full(W1), _full(b2r), _full(b1r), _full(b_resr),
        ],
        out_shapes=(
            jax.ShapeDtypeStruct((N, d), jnp.float32),
            jax.ShapeDtypeStruct((N, d), jnp.float32),
            jax.ShapeDtypeStruct((N, 32), jnp.float32),
        ),
        out_specs=(
            pl.BlockSpec((B, d), lambda i: (i, 0)),
            pl.BlockSpec((B, d), lambda i: (i, 0)),
            pl.BlockSpec((B, 32), lambda i: (i, 0)),
        ),
    )(g1, a2, W2, W1, b2r, b1r, b_resr)

    g2 = _sc_gather(x1, idx).reshape(K, N, 32)

    def mean_body(g_ref, o_ref):
        s = jnp.zeros((B, 32), jnp.float32)
        for k in range(K):
            s = s + g_ref[k]
        o_ref[...] = s * (1.0 / K)

    gg1 = _tc_call(
        mean_body,
        grid=(G,),
        in_specs=[pl.BlockSpec((K, B, 32), lambda i: (0, i, 0))],
        out_shapes=jax.ShapeDtypeStruct((N, 32), jnp.float32),
        out_specs=pl.BlockSpec((B, 32), lambda i: (i, 0)),
    )(g2)

    g3 = _sc_gather(gg1, idx).reshape(K, N, 32)

    def fin_body(g_ref, c_ref, f1_ref, w2_ref, wres_ref, bres_ref, o_ref):
        qsum = jnp.zeros((B, 32), jnp.float32)
        for k in range(K):
            qsum = qsum + g_ref[k]
        q = qsum * (1.0 / K)
        f1 = f1_ref[...]
        o_ref[...] = (c_ref[...] + _DOT(q, w2_ref[...][96:128, :]) + f1
                      + _DOT(f1, wres_ref[...]) + bres_ref[...])

    out = _tc_call(
        fin_body,
        grid=(G,),
        in_specs=[
            pl.BlockSpec((K, B, 32), lambda i: (0, i, 0)),
            pl.BlockSpec((B, d), lambda i: (i, 0)),
            pl.BlockSpec((B, d), lambda i: (i, 0)),
            _full(W2), _full(W_res), _full(b_resr),
        ],
        out_shapes=jax.ShapeDtypeStruct((N, d), jnp.float32),
        out_specs=pl.BlockSpec((B, d), lambda i: (i, 0)),
    )(g3, c, f1, W2, W_res, b_resr)

    return out


# speedup vs baseline: 6.9546x; 1.6537x over previous
import functools

import jax
import jax.numpy as jnp
from jax import lax
from jax.experimental import pallas as pl
from jax.experimental.pallas import tpu as pltpu
from jax.experimental.pallas import tpu_sc as plsc

NC = 2
NS = 16
NW = NC * NS
NPAD = 10240
PPW = NPAD // NW
CH = 64
NCHUNK = PPW // CH
KNN = 16

def _mesh():
    return plsc.VectorSubcoreMesh(
        core_axis_name="c", subcore_axis_name="s", num_cores=NC, num_subcores=NS
    )


_sc_params = pltpu.CompilerParams(use_tc_tiling_on_sc=False, needs_layout_passes=False)


def _wid():
    return lax.axis_index("s") * NC + lax.axis_index("c")


def _sc_sqd(xt_flat, knn_t):
    @functools.partial(
        pl.kernel,
        mesh=_mesh(),
        out_type=jax.ShapeDtypeStruct((NPAD, 32), jnp.float32),
        scratch_types=[
            pltpu.VMEM((3 * NPAD,), jnp.float32),
            pltpu.VMEM((KNN, CH), jnp.int32),
            pltpu.VMEM((CH, 32), jnp.float32),
        ],
        compiler_params=_sc_params,
    )
    def k(xt_hbm, knnt_hbm, out_hbm, table_v, idx_v, stage_v):
        base_pt = _wid() * PPW
        pltpu.sync_copy(xt_hbm, table_v)
        z = jnp.zeros((16,), jnp.float32)

        def zero_row(r, _):
            stage_v[r, pl.ds(0, 16)] = z
            stage_v[r, pl.ds(16, 16)] = z
            return 0

        lax.fori_loop(0, CH, zero_row, 0)
        lane = lax.iota(jnp.int32, 16)

        for ch in range(NCHUNK):
            pt0 = base_pt + ch * CH
            pltpu.sync_copy(knnt_hbm.at[:, pl.ds(pt0, CH)], idx_v)

            def group(j, _):
                i0 = pt0 + 16 * j
                l0 = 16 * j
                xi0 = table_v[pl.ds(i0, 16)]
                xi1 = table_v[pl.ds(NPAD + i0, 16)]
                xi2 = table_v[pl.ds(2 * NPAD + i0, 16)]
                row_idx = l0 + lane
                ds0 = z
                ds1 = z
                ds2 = z
                for kk in range(KNN):
                    nb = idx_v[kk, pl.ds(l0, 16)]
                    v0 = plsc.load_gather(table_v, [nb])
                    v1 = plsc.load_gather(table_v, [nb + NPAD])
                    v2 = plsc.load_gather(table_v, [nb + 2 * NPAD])
                    d0 = xi0 - v0
                    d1 = xi1 - v1
                    d2 = xi2 - v2
                    sq = d0 * d0 + d1 * d1 + d2 * d2
                    ds0 = ds0 + d0
                    ds1 = ds1 + d1
                    ds2 = ds2 + d2
                    plsc.store_scatter(
                        stage_v, [row_idx, jnp.full((16,), kk, jnp.int32)], sq
                    )
                plsc.store_scatter(
                    stage_v, [row_idx, jnp.full((16,), 16, jnp.int32)], ds0
                )
                plsc.store_scatter(
                    stage_v, [row_idx, jnp.full((16,), 17, jnp.int32)], ds1
                )
                plsc.store_scatter(
                    stage_v, [row_idx, jnp.full((16,), 18, jnp.int32)], ds2
                )
                return 0

            lax.fori_loop(0, CH // 16, group, 0)
            pltpu.sync_copy(stage_v, out_hbm.at[pl.ds(pt0, CH)])

    return k(xt_flat, knn_t)


def _sc_gsum(table, idx_flat):
    CHF = CH * KNN

    @functools.partial(
        pl.kernel,
        mesh=_mesh(),
        out_type=jax.ShapeDtypeStruct((NPAD, 32), jnp.float32),
        scratch_types=[
            pltpu.VMEM((CHF,), jnp.int32),
            pltpu.VMEM((CHF,), jnp.int32),
            pltpu.VMEM((CHF, 32), jnp.float32),
            pltpu.VMEM((CHF, 32), jnp.float32),
            pltpu.VMEM((CH, 32), jnp.float32),
            pltpu.SemaphoreType.DMA,
            pltpu.SemaphoreType.DMA,
        ],
        compiler_params=_sc_params,
    )
    def k(table_hbm, idx_hbm, out_hbm, idx0, idx1, rows0, rows1, stage_v,
          sem0, sem1):
        basep = _wid() * PPW
        basef = basep * KNN
        idx_v = (idx0, idx1)
        rows_v = (rows0, rows1)
        sems = (sem0, sem1)

        pltpu.sync_copy(idx_hbm.at[pl.ds(basef, CHF)], idx0)
        cps = [pltpu.async_copy(table_hbm.at[idx0], rows0, sem0), None]

        for ch in range(NCHUNK):
            cur = ch % 2
            nxt = 1 - cur
            if ch + 1 < NCHUNK:
                pltpu.sync_copy(
                    idx_hbm.at[pl.ds(basef + (ch + 1) * CHF, CHF)], idx_v[nxt]
                )
                cps[nxt] = pltpu.async_copy(
                    table_hbm.at[idx_v[nxt]], rows_v[nxt], sems[nxt]
                )
            cps[cur].wait()
            rows = rows_v[cur]

            def point(pp, _):
                r0 = pp * KNN
                a0 = rows[r0, pl.ds(0, 16)]
                a1 = rows[r0, pl.ds(16, 16)]
                for kk in range(1, KNN):
                    a0 = a0 + rows[r0 + kk, pl.ds(0, 16)]
                    a1 = a1 + rows[r0 + kk, pl.ds(16, 16)]
                stage_v[pp, pl.ds(0, 16)] = a0
                stage_v[pp, pl.ds(16, 16)] = a1
                return 0

            lax.fori_loop(0, CH, point, 0)
            pltpu.sync_copy(stage_v, out_hbm.at[pl.ds(basep + ch * CH, CH)])

    return k(table, idx_flat)


def _full(a):
    return pl.BlockSpec(a.shape, lambda i: (0,) * a.ndim)


_DOT = functools.partial(jnp.dot, precision=jax.lax.Precision.HIGHEST)

_B = 1024
_G = NPAD // _B


def kernel(inputs, knn, W_res, b_res, W1, b1, Wl1, bl1, Wl2, bl2, W2, b2):
    N, K = knn.shape
    d = W_res.shape[0]

    knn_pad = jnp.pad(knn.astype(jnp.int32), ((0, NPAD - N), (0, 0)))
    knn_t = knn_pad.T.copy()
    idx_flat = knn_pad.reshape(-1)
    xt_flat = jnp.pad(inputs, ((0, NPAD - N), (0, 0))).T.reshape(-1)

    Wl1p = jnp.pad(Wl1[0:3], ((0, 13), (0, 0)))
    Wl2p = jnp.pad(Wl2[0:3], ((0, 13), (0, 0)))
    Wl1r3 = Wl1[3:4]
    Wl2r3 = Wl2[3:4]
    bl1r = bl1[None, :]
    bl2r = bl2[None, :]
    b1r = b1[None, :]
    b2r = b2[None, :]
    b_resr = b_res[None, :]

    sqd = _sc_sqd(xt_flat, knn_t)

    def gf_body(s_ref, wl1_ref, wl2_ref, wr1_ref, wr2_ref, bl1_ref, bl2_ref,
                a1_ref, a2_ref):
        s = s_ref[...]
        nsum = jnp.sum(jnp.sqrt(s[:, 0:16]), axis=1, keepdims=True)
        dsum = s[:, 16:32]
        inv_k = 1.0 / K
        a1_ref[...] = (_DOT(dsum, wl1_ref[...]) + nsum * wr1_ref[...]) * inv_k \
            + bl1_ref[...]
        a2_ref[...] = (_DOT(dsum, wl2_ref[...]) + nsum * wr2_ref[...]) * inv_k \
            + bl2_ref[...]

    a1, a2 = pl.pallas_call(
        gf_body,
        grid=(_G,),
        in_specs=[
            pl.BlockSpec((_B, 32), lambda i: (i, 0)),
            _full(Wl1p), _full(Wl2p), _full(Wl1r3), _full(Wl2r3),
            _full(bl1r), _full(bl2r),
        ],
        out_shape=(
            jax.ShapeDtypeStruct((NPAD, 32), jnp.float32),
            jax.ShapeDtypeStruct((NPAD, 64), jnp.float32),
        ),
        out_specs=(
            pl.BlockSpec((_B, 32), lambda i: (i, 0)),
            pl.BlockSpec((_B, 64), lambda i: (i, 0)),
        ),
    )(sqd, Wl1p, Wl2p, Wl1r3, Wl2r3, bl1r, bl2r)

    p_s = _sc_gsum(a1, idx_flat)

    def c_body(ps_ref, a2_ref, w2_ref, w1_ref, b2_ref, b1_ref, bres_ref,
               c_ref, f1_ref, x1_ref):
        w2 = w2_ref[...]
        b1v = b1_ref[...]
        c = (_DOT(a2_ref[...], w2[0:64, :])
             + _DOT(ps_ref[...] * (1.0 / K), w2[64:96, :]) + b2_ref[...])
        row = _DOT(b1v, w2[96:128, :]) + bres_ref[...]
        f1 = c + row
        c_ref[...] = c
        f1_ref[...] = f1
        x1_ref[...] = _DOT(f1, w1_ref[...]) + b1v

    c, f1, x1 = pl.pallas_call(
        c_body,
        grid=(_G,),
        in_specs=[
            pl.BlockSpec((_B, 32), lambda i: (i, 0)),
            pl.BlockSpec((_B, 64), lambda i: (i, 0)),
            _full(W2), _full(W1), _full(b2r), _full(b1r), _full(b_resr),
        ],
        out_shape=(
            jax.ShapeDtypeStruct((NPAD, d), jnp.float32),
            jax.ShapeDtypeStruct((NPAD, d), jnp.float32),
            jax.ShapeDtypeStruct((NPAD, 32), jnp.float32),
        ),
        out_specs=(
            pl.BlockSpec((_B, d), lambda i: (i, 0)),
            pl.BlockSpec((_B, d), lambda i: (i, 0)),
            pl.BlockSpec((_B, 32), lambda i: (i, 0)),
        ),
    )(p_s, a2, W2, W1, b2r, b1r, b_resr)

    g1_s = _sc_gsum(x1, idx_flat)
    q_s = _sc_gsum(g1_s, idx_flat)

    def fin_body(qs_ref, c_ref, f1_ref, w2_ref, wres_ref, bres_ref, o_ref):
        f1v = f1_ref[...]
        q = qs_ref[...] * (1.0 / (K * K))
        o_ref[...] = (c_ref[...] + _DOT(q, w2_ref[...][96:128, :]) + f1v
                      + _DOT(f1v, wres_ref[...]) + bres_ref[...])

    out = pl.pallas_call(
        fin_body,
        grid=(_G,),
        in_specs=[
            pl.BlockSpec((_B, 32), lambda i: (i, 0)),
            pl.BlockSpec((_B, d), lambda i: (i, 0)),
            pl.BlockSpec((_B, d), lambda i: (i, 0)),
            _full(W2), _full(W_res), _full(b_resr),
        ],
        out_shape=jax.ShapeDtypeStruct((NPAD, d), jnp.float32),
        out_specs=pl.BlockSpec((_B, d), lambda i: (i, 0)),
    )(q_s, c, f1, W2, W_res, b_resr)

    return out[:N]
